# baseline (speedup 1.0000x reference)
import functools

import jax
import jax.numpy as jnp
from jax import lax
from jax.experimental import pallas as pl
from jax.experimental.pallas import tpu as pltpu
from jax.experimental.pallas import tpu_sc as plsc

N = 10000
E = 320000
NP = 10240
G = 64
NC = 2
NS = 16
NW = NC * NS
EPT = E // NW
CH = 128
NCHT = 2560
EP = NCHT * CH
CPT = NCHT // NW
RPT = NP // NS

_mesh = functools.partial(
    plsc.VectorSubcoreMesh, core_axis_name="c", subcore_axis_name="s"
)


def _deg_body(dst_hbm, out_hbm, idx_v, deg_v):
    c = lax.axis_index("c")
    s = lax.axis_index("s")
    wid = c * NS + s

    def zero(i, _):
        deg_v[pl.ds(i * 16, 16)] = jnp.zeros((16,), jnp.float32)
        return 0

    lax.fori_loop(0, NP // 16, zero, 0)

    pltpu.sync_copy(dst_hbm.at[pl.ds(wid * EPT, EPT)], idx_v)
    ones = jnp.ones((16,), jnp.float32)

    def body(j, _):
        idx = idx_v[pl.ds(j * 16, 16)]
        plsc.addupdate_scatter(deg_v, [idx], ones)
        return 0

    lax.fori_loop(0, EPT // 16, body, 0)
    pltpu.sync_copy(deg_v, out_hbm.at[wid])


def _deg_call(dst):
    return pl.kernel(
        _deg_body,
        out_type=jax.ShapeDtypeStruct((NW, NP), jnp.float32),
        mesh=_mesh(),
        scratch_types=[
            pltpu.VMEM((EPT,), jnp.int32),
            pltpu.VMEM((NP,), jnp.float32),
        ],
        compiler_params=pltpu.CompilerParams(needs_layout_passes=False),
    )(dst)


def _spmm_body(F, NB, preload, xs_hbm, src_hbm, dst_hbm, out_hbm, sidx, didx, *rest):
    rows = rest[:NB]
    gsem = rest[NB]
    ssem = rest[NB + 1]
    acc = rest[NB + 2]
    c = lax.axis_index("c")
    s = lax.axis_index("s")
    wid = c * NS + s
    base = wid * CPT
    NG = CPT // NB

    def zr(r, _):
        def zc(k, _):
            rows[0][r, pl.ds(k * 16, 16)] = jnp.zeros((16,), jnp.float32)
            return 0

        lax.fori_loop(0, F // 16, zc, 0)
        return 0

    lax.fori_loop(0, CH, zr, 0)
    for j in range(RPT // CH):
        pltpu.sync_copy(rows[0], acc.at[pl.ds(s * RPT + j * CH, CH)])

    if preload:
        pltpu.sync_copy(src_hbm.at[pl.ds(base, CPT)], sidx)
        pltpu.sync_copy(dst_hbm.at[pl.ds(base, CPT)], didx)

        def ldidx(i, b):
            del i, b

        def srow(i, b):
            del b
            return sidx.at[i]

        def drow(i, b):
            del b
            return didx.at[i]

    else:
        def ldidx(i, b):
            pltpu.sync_copy(src_hbm.at[pl.ds(base + i, 1)], sidx.at[pl.ds(b, 1)])
            pltpu.sync_copy(dst_hbm.at[pl.ds(base + i, 1)], didx.at[pl.ds(b, 1)])

        def srow(i, b):
            del i
            return sidx.at[b]

        def drow(i, b):
            del i
            return didx.at[b]

    def gstart(i, b):
        pltpu.async_copy(xs_hbm.at[srow(i, b)], rows[b], gsem.at[b])

    def gwait(b):
        pltpu.make_async_copy(xs_hbm.at[srow(0, b)], rows[b], gsem.at[b]).wait()

    def sstart(i, b):
        pltpu.async_copy(rows[b], acc.at[drow(i, b)], ssem.at[b], add=True)

    def swait(b):
        pltpu.make_async_copy(rows[b], acc.at[drow(0, b)], ssem.at[b]).wait()

    for b in range(NB):
        ldidx(b, b)
        gstart(b, b)
    plsc.subcore_barrier()
    for b in range(NB):
        gwait(b)
        sstart(b, b)

    def group(g, _):
        i0 = g * NB
        for b in range(NB):
            swait(b)
            ldidx(i0 + b, b)
            gstart(i0 + b, b)
        for b in range(NB):
            gwait(b)
            sstart(i0 + b, b)
        return 0

    lax.fori_loop(1, NG, group, 0)
    for b in range(NB):
        swait(b)

    plsc.subcore_barrier()
    pltpu.sync_copy(
        acc.at[pl.ds(s * RPT, RPT)], out_hbm.at[c, pl.ds(s * RPT, RPT)]
    )


def _spmm_call(F, NB, preload, xs, src2d, dst2d):
    nidx = CPT if preload else NB
    return pl.kernel(
        functools.partial(_spmm_body, F, NB, preload),
        out_type=jax.ShapeDtypeStruct((NC, NP, F), jnp.float32),
        mesh=_mesh(),
        scratch_types=[
            pltpu.VMEM((nidx, CH), jnp.int32),
            pltpu.VMEM((nidx, CH), jnp.int32),
        ]
        + [pltpu.VMEM((CH, F), jnp.float32) for _ in range(NB)]
        + [
            pltpu.SemaphoreType.DMA((NB,)),
            pltpu.SemaphoreType.DMA((NB,)),
            pltpu.VMEM_SHARED((NP, F), jnp.float32),
        ],
        compiler_params=pltpu.CompilerParams(use_tc_tiling_on_sc=False),
    )(xs, src2d, dst2d)


def _dinv_body(degp_ref, dinv_ref):
    deg = 1.0 + jnp.sum(degp_ref[...], axis=0, keepdims=True)
    dinv_ref[...] = lax.rsqrt(jnp.maximum(deg, 1e-12))


def _scale_body(x_ref, d_ref, o_ref):
    o_ref[...] = x_ref[...] * d_ref[...]


def _mid_body(p0, p1, xs, d, w1, b1, w2, o):
    agg = d[...] * (p0[...] + p1[...] + xs[...])
    h1 = jnp.maximum(
        jnp.dot(agg, w1[...], preferred_element_type=jnp.float32) + b1[...], 0.0
    )
    g = jnp.dot(h1, w2[...], preferred_element_type=jnp.float32)
    o[...] = g * d[...]


def _head_body(q0, q1, gs, d, b2, bt, fc1w, fc1b, fc2w, fc2b, o):
    h2 = jnp.maximum(d[...] * (q0[...] + q1[...] + gs[...]) + b2[...], 0.0)
    gid = lax.broadcasted_iota(jnp.int32, (G, NP), 0)
    oh = (gid == bt[...]).astype(jnp.float32)
    psum = jnp.dot(oh, h2, preferred_element_type=jnp.float32)
    cnt = jnp.sum(oh, axis=1, keepdims=True)
    pooled = psum / jnp.maximum(cnt, 1.0)
    z = jnp.tanh(jnp.dot(pooled, fc1w[...], preferred_element_type=jnp.float32) + fc1b[...])
    zz = jnp.dot(z, fc2w[...], preferred_element_type=jnp.float32) + fc2b[...]
    o[...] = jax.nn.sigmoid(zz)


def kernel(x, edge_index, batch, W1, b1, W2, b2, fc1_w, fc1_b, fc2_w, fc2_b):
    f32 = jnp.float32
    src = edge_index[0].astype(jnp.int32)
    dst = edge_index[1].astype(jnp.int32)
    src2d = jnp.pad(src, (0, EP - E), constant_values=N).reshape(NCHT, CH)
    dst2d = jnp.pad(dst, (0, EP - E), constant_values=N).reshape(NCHT, CH)
    x_pad = jnp.pad(x.astype(f32), ((0, NP - N), (0, 0)))
    batch_pad = jnp.pad(
        batch.astype(jnp.int32), (0, NP - N), constant_values=2**20
    ).reshape(1, NP)

    degp = _deg_call(dst)

    dinv_row = pl.pallas_call(
        _dinv_body,
        out_shape=jax.ShapeDtypeStruct((1, NP), f32),
    )(degp)
    dinv_col = dinv_row.reshape(NP, 1)

    RB = 1280
    xs = pl.pallas_call(
        _scale_body,
        grid=(NP // RB,),
        in_specs=[
            pl.BlockSpec((RB, 128), lambda i: (i, 0)),
            pl.BlockSpec((RB, 1), lambda i: (i, 0)),
        ],
        out_specs=pl.BlockSpec((RB, 128), lambda i: (i, 0)),
        out_shape=jax.ShapeDtypeStruct((NP, 128), f32),
    )(x_pad, dinv_col)

    P = _spmm_call(128, 2, False, xs, src2d, dst2d)

    gs = pl.pallas_call(
        _mid_body,
        grid=(NP // RB,),
        in_specs=[
            pl.BlockSpec((RB, 128), lambda i: (i, 0)),
            pl.BlockSpec((RB, 128), lambda i: (i, 0)),
            pl.BlockSpec((RB, 128), lambda i: (i, 0)),
            pl.BlockSpec((RB, 1), lambda i: (i, 0)),
            pl.BlockSpec((128, 128), lambda i: (0, 0)),
            pl.BlockSpec((1, 128), lambda i: (0, 0)),
            pl.BlockSpec((128, 32), lambda i: (0, 0)),
        ],
        out_specs=pl.BlockSpec((RB, 32), lambda i: (i, 0)),
        out_shape=jax.ShapeDtypeStruct((NP, 32), f32),
    )(P[0], P[1], xs, dinv_col, W1, b1.reshape(1, 128), W2)

    Q = _spmm_call(32, 8, True, gs, src2d, dst2d)

    out = pl.pallas_call(
        _head_body,
        out_shape=jax.ShapeDtypeStruct((G, 1), f32),
    )(
        Q[0],
        Q[1],
        gs,
        dinv_col,
        b2.reshape(1, 32),
        batch_pad,
        fc1_w,
        fc1_b.reshape(1, 16),
        fc2_w,
        fc2_b.reshape(1, 1),
    )
    return out


# speedup vs baseline: 1.6282x; 1.6282x over previous
import functools

import jax
import jax.numpy as jnp
from jax import lax
from jax.experimental import pallas as pl
from jax.experimental.pallas import tpu as pltpu
from jax.experimental.pallas import tpu_sc as plsc

N = 10000
E = 320000
NP = 10240
G = 64
NC = 2
NS = 16
NW = NC * NS
EPT = E // NW
CH = 128
NCHK = E // CH
CPW = NCHK // NW
CREM = NCHK - CPW * NW
RPT = NP // NS

_mesh = functools.partial(
    plsc.VectorSubcoreMesh, core_axis_name="c", subcore_axis_name="s"
)


def _deg_body(dst_hbm, out_hbm, idx_v, deg_v):
    c = lax.axis_index("c")
    s = lax.axis_index("s")
    wid = c * NS + s

    def zero(i, _):
        deg_v[pl.ds(i * 16, 16)] = jnp.zeros((16,), jnp.float32)
        return 0

    lax.fori_loop(0, NP // 16, zero, 0)

    pltpu.sync_copy(dst_hbm.at[pl.ds(wid * EPT, EPT)], idx_v)
    ones = jnp.ones((16,), jnp.float32)

    def body(j, _):
        idx = idx_v[pl.ds(j * 16, 16)]
        plsc.addupdate_scatter(deg_v, [idx], ones)
        return 0

    lax.fori_loop(0, EPT // 16, body, 0)
    pltpu.sync_copy(deg_v, out_hbm.at[wid])


def _deg_call(dst):
    return pl.kernel(
        _deg_body,
        out_type=jax.ShapeDtypeStruct((NW, NP), jnp.float32),
        mesh=_mesh(),
        scratch_types=[
            pltpu.VMEM((EPT,), jnp.int32),
            pltpu.VMEM((NP,), jnp.float32),
        ],
        compiler_params=pltpu.CompilerParams(needs_layout_passes=False),
    )(dst)


def _spmm1_body(xs_hbm, src_hbm, dst_hbm, out_hbm, idx_s, idx_d, rows, acc, sem):
    F = 128
    c = lax.axis_index("c")
    s = lax.axis_index("s")
    wid = c * NS + s

    def zr(r, _):
        def zc(k, _):
            rows[r, pl.ds(k * 16, 16)] = jnp.zeros((16,), jnp.float32)
            return 0

        lax.fori_loop(0, F // 16, zc, 0)
        return 0

    lax.fori_loop(0, CH, zr, 0)
    for j in range(RPT // CH):
        pltpu.sync_copy(rows, acc.at[pl.ds(s * RPT + j * CH, CH)])
    plsc.subcore_barrier()

    base = wid * CPW + jnp.minimum(wid, CREM)
    n = jnp.where(wid < CREM, CPW + 1, CPW)

    def edge(i, _):
        off = (base + i) * CH
        pltpu.sync_copy(src_hbm.at[pl.ds(off, CH)], idx_s)
        pltpu.async_copy(xs_hbm.at[idx_s], rows, sem).wait()
        pltpu.sync_copy(dst_hbm.at[pl.ds(off, CH)], idx_d)
        pltpu.sync_copy(rows, acc.at[idx_d], add=True)
        return 0

    lax.fori_loop(0, n, edge, 0)
    plsc.subcore_barrier()
    pltpu.sync_copy(
        acc.at[pl.ds(s * RPT, RPT)], out_hbm.at[c, pl.ds(s * RPT, RPT)]
    )


def _spmm1_call(xs, src, dst):
    return pl.kernel(
        _spmm1_body,
        out_type=jax.ShapeDtypeStruct((NC, NP, 128), jnp.float32),
        mesh=_mesh(),
        scratch_types=[
            pltpu.VMEM((CH,), jnp.int32),
            pltpu.VMEM((CH,), jnp.int32),
            pltpu.VMEM((CH, 128), jnp.float32),
            pltpu.VMEM_SHARED((NP, 128), jnp.float32),
            pltpu.SemaphoreType.DMA,
        ],
        compiler_params=pltpu.CompilerParams(use_tc_tiling_on_sc=False),
    )(xs, src, dst)


def _spmm2_body(gs_hbm, src_hbm, dst_hbm, out_hbm, sidx, didx, rows, acc, sem):
    F = 32
    c = lax.axis_index("c")
    s = lax.axis_index("s")
    wid = c * NS + s

    def zr(r, _):
        def zc(k, _):
            rows[r, pl.ds(k * 16, 16)] = jnp.zeros((16,), jnp.float32)
            return 0

        lax.fori_loop(0, F // 16, zc, 0)
        return 0

    lax.fori_loop(0, CH, zr, 0)
    for j in range(RPT // CH):
        pltpu.sync_copy(rows, acc.at[pl.ds(s * RPT + j * CH, CH)])
    plsc.subcore_barrier()

    base = wid * CPW + jnp.minimum(wid, CREM)
    n = jnp.where(wid < CREM, CPW + 1, CPW)
    pltpu.sync_copy(src_hbm.at[pl.ds(base, CPW + 1)], sidx)
    pltpu.sync_copy(dst_hbm.at[pl.ds(base, CPW + 1)], didx)

    def edge(i, _):
        pltpu.async_copy(gs_hbm.at[sidx.at[i]], rows, sem).wait()
        pltpu.sync_copy(rows, acc.at[didx.at[i]], add=True)
        return 0

    lax.fori_loop(0, n, edge, 0)
    plsc.subcore_barrier()
    pltpu.sync_copy(
        acc.at[pl.ds(s * RPT, RPT)], out_hbm.at[c, pl.ds(s * RPT, RPT)]
    )


def _spmm2_call(gs, src, dst):
    return pl.kernel(
        _spmm2_body,
        out_type=jax.ShapeDtypeStruct((NC, NP, 32), jnp.float32),
        mesh=_mesh(),
        scratch_types=[
            pltpu.VMEM((CPW + 1, CH), jnp.int32),
            pltpu.VMEM((CPW + 1, CH), jnp.int32),
            pltpu.VMEM((CH, 32), jnp.float32),
            pltpu.VMEM_SHARED((NP, 32), jnp.float32),
            pltpu.SemaphoreType.DMA,
        ],
        compiler_params=pltpu.CompilerParams(use_tc_tiling_on_sc=False),
    )(gs, src, dst)


def _dinv_body(degp_ref, dinv_ref):
    deg = 1.0 + jnp.sum(degp_ref[...], axis=0, keepdims=True)
    dinv_ref[...] = lax.rsqrt(jnp.maximum(deg, 1e-12))


def _scale_body(x_ref, d_ref, o_ref):
    o_ref[...] = x_ref[...] * d_ref[...]


def _mid_body(p0, p1, xs, d, w1, b1, w2, o):
    agg = d[...] * (p0[...] + p1[...] + xs[...])
    h1 = jnp.maximum(
        jnp.dot(agg, w1[...], preferred_element_type=jnp.float32) + b1[...], 0.0
    )
    g = jnp.dot(h1, w2[...], preferred_element_type=jnp.float32)
    o[...] = g * d[...]


def _head_body(q0, q1, gs, d, b2, bt, fc1w, fc1b, fc2w, fc2b, o):
    h2 = jnp.maximum(d[...] * (q0[...] + q1[...] + gs[...]) + b2[...], 0.0)
    gid = lax.broadcasted_iota(jnp.int32, (G, NP), 0)
    oh = (gid == bt[...]).astype(jnp.float32)
    psum = jnp.dot(oh, h2, preferred_element_type=jnp.float32)
    cnt = jnp.sum(oh, axis=1, keepdims=True)
    pooled = psum / jnp.maximum(cnt, 1.0)
    z = jnp.tanh(jnp.dot(pooled, fc1w[...], preferred_element_type=jnp.float32) + fc1b[...])
    zz = jnp.dot(z, fc2w[...], preferred_element_type=jnp.float32) + fc2b[...]
    o[...] = jax.nn.sigmoid(zz)


def kernel(x, edge_index, batch, W1, b1, W2, b2, fc1_w, fc1_b, fc2_w, fc2_b):
    f32 = jnp.float32
    src = edge_index[0].astype(jnp.int32)
    dst = edge_index[1].astype(jnp.int32)
    npad = (NCHK + CREM) * CH - E
    src2d = jnp.pad(src, (0, npad), constant_values=N).reshape(NCHK + CREM, CH)
    dst2d = jnp.pad(dst, (0, npad), constant_values=N).reshape(NCHK + CREM, CH)
    x_pad = jnp.pad(x.astype(f32), ((0, NP - N), (0, 0)))
    batch_pad = jnp.pad(
        batch.astype(jnp.int32), (0, NP - N), constant_values=2**20
    ).reshape(1, NP)

    degp = _deg_call(dst)

    dinv_row = pl.pallas_call(
        _dinv_body,
        out_shape=jax.ShapeDtypeStruct((1, NP), f32),
    )(degp)
    dinv_col = dinv_row.reshape(NP, 1)

    RB = 1280
    xs = pl.pallas_call(
        _scale_body,
        grid=(NP // RB,),
        in_specs=[
            pl.BlockSpec((RB, 128), lambda i: (i, 0)),
            pl.BlockSpec((RB, 1), lambda i: (i, 0)),
        ],
        out_specs=pl.BlockSpec((RB, 128), lambda i: (i, 0)),
        out_shape=jax.ShapeDtypeStruct((NP, 128), f32),
    )(x_pad, dinv_col)

    P = _spmm1_call(xs, src, dst)

    gs = pl.pallas_call(
        _mid_body,
        grid=(NP // RB,),
        in_specs=[
            pl.BlockSpec((RB, 128), lambda i: (i, 0)),
            pl.BlockSpec((RB, 128), lambda i: (i, 0)),
            pl.BlockSpec((RB, 128), lambda i: (i, 0)),
            pl.BlockSpec((RB, 1), lambda i: (i, 0)),
            pl.BlockSpec((128, 128), lambda i: (0, 0)),
            pl.BlockSpec((1, 128), lambda i: (0, 0)),
            pl.BlockSpec((128, 32), lambda i: (0, 0)),
        ],
        out_specs=pl.BlockSpec((RB, 32), lambda i: (i, 0)),
        out_shape=jax.ShapeDtypeStruct((NP, 32), f32),
    )(P[0], P[1], xs, dinv_col, W1, b1.reshape(1, 128), W2)

    Q = _spmm2_call(gs, src2d, dst2d)

    out = pl.pallas_call(
        _head_body,
        out_shape=jax.ShapeDtypeStruct((G, 1), f32),
    )(
        Q[0],
        Q[1],
        gs,
        dinv_col,
        b2.reshape(1, 32),
        batch_pad,
        fc1_w,
        fc1_b.reshape(1, 16),
        fc2_w,
        fc2_b.reshape(1, 1),
    )
    return out


# speedup vs baseline: 1.9499x; 1.1976x over previous
import functools

import jax
import jax.numpy as jnp
from jax import lax
from jax.experimental import pallas as pl
from jax.experimental.pallas import tpu as pltpu
from jax.experimental.pallas import tpu_sc as plsc

N = 10000
E = 320000
NP = 10240
G = 64
NC = 2
NS = 16
NW = NC * NS
EPT = E // NW
CH = 128
NCHK = E // CH
CPW = NCHK // NW
CREM = NCHK - CPW * NW
RPT = NP // NS

_mesh = functools.partial(
    plsc.VectorSubcoreMesh, core_axis_name="c", subcore_axis_name="s"
)


def _deg_body(dst_hbm, out_hbm, idx_v, deg_v):
    c = lax.axis_index("c")
    s = lax.axis_index("s")
    wid = c * NS + s

    def zero(i, _):
        deg_v[pl.ds(i * 16, 16)] = jnp.zeros((16,), jnp.float32)
        return 0

    lax.fori_loop(0, NP // 16, zero, 0)

    pltpu.sync_copy(dst_hbm.at[pl.ds(wid * EPT, EPT)], idx_v)
    ones = jnp.ones((16,), jnp.float32)

    def body(j, _):
        idx = idx_v[pl.ds(j * 16, 16)]
        plsc.addupdate_scatter(deg_v, [idx], ones)
        return 0

    lax.fori_loop(0, EPT // 16, body, 0)
    pltpu.sync_copy(deg_v, out_hbm.at[wid])


def _deg_call(dst):
    return pl.kernel(
        _deg_body,
        out_type=jax.ShapeDtypeStruct((NW, NP), jnp.float32),
        mesh=_mesh(),
        scratch_types=[
            pltpu.VMEM((EPT,), jnp.int32),
            pltpu.VMEM((NP,), jnp.float32),
        ],
        compiler_params=pltpu.CompilerParams(needs_layout_passes=False),
    )(dst)


def _spmm_body(F, wide_scat, gs_hbm, src_hbm, dst1_hbm, out_hbm, sidx, didx, rows, acc, sem):
    c = lax.axis_index("c")
    s = lax.axis_index("s")
    wid = c * NS + s

    def zr(r, _):
        def zc(k, _):
            rows[r, pl.ds(k * 16, 16)] = jnp.zeros((16,), jnp.float32)
            return 0

        lax.fori_loop(0, F // 16, zc, 0)
        return 0

    lax.fori_loop(0, CH, zr, 0)
    for j in range(RPT // CH):
        pltpu.sync_copy(rows, acc.at[pl.ds(s * RPT + j * CH, CH)])
    plsc.subcore_barrier()

    base = wid * CPW + jnp.minimum(wid, CREM)
    n = jnp.where(wid < CREM, CPW + 1, CPW)
    pltpu.sync_copy(src_hbm.at[pl.ds(base, CPW + 1)], sidx)
    if not wide_scat:
        pltpu.sync_copy(dst1_hbm.at[pl.ds(base, CPW + 1)], didx)

    def edge(i, _):
        dma = pltpu.async_copy(gs_hbm.at[sidx.at[i]], rows, sem)
        if wide_scat:
            pltpu.sync_copy(dst1_hbm.at[pl.ds((base + i) * CH, CH)], didx)
            dma.wait()
            pltpu.sync_copy(rows, acc.at[didx], add=True)
        else:
            dma.wait()
            pltpu.sync_copy(rows, acc.at[didx.at[i]], add=True)
        return 0

    lax.fori_loop(0, n, edge, 0)
    plsc.subcore_barrier()
    pltpu.sync_copy(
        acc.at[pl.ds(s * RPT, RPT)], out_hbm.at[c, pl.ds(s * RPT, RPT)]
    )


def _spmm_call(F, wide_scat, tbl, src2d, dst):
    return pl.kernel(
        functools.partial(_spmm_body, F, wide_scat),
        out_type=jax.ShapeDtypeStruct((NC, NP, F), jnp.float32),
        mesh=_mesh(),
        scratch_types=[
            pltpu.VMEM((CPW + 1, CH), jnp.int32),
            pltpu.VMEM((CH,), jnp.int32)
            if wide_scat
            else pltpu.VMEM((CPW + 1, CH), jnp.int32),
            pltpu.VMEM((CH, F), jnp.float32),
            pltpu.VMEM_SHARED((NP, F), jnp.float32),
            pltpu.SemaphoreType.DMA,
        ],
        compiler_params=pltpu.CompilerParams(use_tc_tiling_on_sc=False),
    )(tbl, src2d, dst)


def _dinv_body(degp_ref, dinv_ref):
    deg = 1.0 + jnp.sum(degp_ref[...], axis=0, keepdims=True)
    dinv_ref[...] = lax.rsqrt(jnp.maximum(deg, 1e-12))


def _scale_body(x_ref, d_ref, o_ref):
    o_ref[...] = x_ref[...] * d_ref[...]


def _mid_body(p0, p1, xs, d, w1, b1, w2, o):
    agg = d[...] * (p0[...] + p1[...] + xs[...])
    h1 = jnp.maximum(
        jnp.dot(agg, w1[...], preferred_element_type=jnp.float32) + b1[...], 0.0
    )
    g = jnp.dot(h1, w2[...], preferred_element_type=jnp.float32)
    o[...] = g * d[...]


def _head_body(q0, q1, gs, d, b2, bt, fc1w, fc1b, fc2w, fc2b, o):
    h2 = jnp.maximum(d[...] * (q0[...] + q1[...] + gs[...]) + b2[...], 0.0)
    gid = lax.broadcasted_iota(jnp.int32, (G, NP), 0)
    oh = (gid == bt[...]).astype(jnp.float32)
    psum = jnp.dot(oh, h2, preferred_element_type=jnp.float32)
    cnt = jnp.sum(oh, axis=1, keepdims=True)
    pooled = psum / jnp.maximum(cnt, 1.0)
    z = jnp.tanh(jnp.dot(pooled, fc1w[...], preferred_element_type=jnp.float32) + fc1b[...])
    zz = jnp.dot(z, fc2w[...], preferred_element_type=jnp.float32) + fc2b[...]
    o[...] = jax.nn.sigmoid(zz)


def kernel(x, edge_index, batch, W1, b1, W2, b2, fc1_w, fc1_b, fc2_w, fc2_b):
    f32 = jnp.float32
    src = edge_index[0].astype(jnp.int32)
    dst = edge_index[1].astype(jnp.int32)
    npad = (NCHK + CREM) * CH - E
    src2d = jnp.pad(src, (0, npad), constant_values=N).reshape(NCHK + CREM, CH)
    dst2d = jnp.pad(dst, (0, npad), constant_values=N).reshape(NCHK + CREM, CH)
    x_pad = jnp.pad(x.astype(f32), ((0, NP - N), (0, 0)))
    batch_pad = jnp.pad(
        batch.astype(jnp.int32), (0, NP - N), constant_values=2**20
    ).reshape(1, NP)

    degp = _deg_call(dst)

    dinv_row = pl.pallas_call(
        _dinv_body,
        out_shape=jax.ShapeDtypeStruct((1, NP), f32),
    )(degp)
    dinv_col = dinv_row.reshape(NP, 1)

    RB = 1280
    xs = pl.pallas_call(
        _scale_body,
        grid=(NP // RB,),
        in_specs=[
            pl.BlockSpec((RB, 128), lambda i: (i, 0)),
            pl.BlockSpec((RB, 1), lambda i: (i, 0)),
        ],
        out_specs=pl.BlockSpec((RB, 128), lambda i: (i, 0)),
        out_shape=jax.ShapeDtypeStruct((NP, 128), f32),
    )(x_pad, dinv_col)

    P = _spmm_call(128, True, xs, src2d, dst)

    gs = pl.pallas_call(
        _mid_body,
        grid=(NP // RB,),
        in_specs=[
            pl.BlockSpec((RB, 128), lambda i: (i, 0)),
            pl.BlockSpec((RB, 128), lambda i: (i, 0)),
            pl.BlockSpec((RB, 128), lambda i: (i, 0)),
            pl.BlockSpec((RB, 1), lambda i: (i, 0)),
            pl.BlockSpec((128, 128), lambda i: (0, 0)),
            pl.BlockSpec((1, 128), lambda i: (0, 0)),
            pl.BlockSpec((128, 32), lambda i: (0, 0)),
        ],
        out_specs=pl.BlockSpec((RB, 32), lambda i: (i, 0)),
        out_shape=jax.ShapeDtypeStruct((NP, 32), f32),
    )(P[0], P[1], xs, dinv_col, W1, b1.reshape(1, 128), W2)

    Q = _spmm_call(32, False, gs, src2d, dst2d)

    out = pl.pallas_call(
        _head_body,
        out_shape=jax.ShapeDtypeStruct((G, 1), f32),
    )(
        Q[0],
        Q[1],
        gs,
        dinv_col,
        b2.reshape(1, 32),
        batch_pad,
        fc1_w,
        fc1_b.reshape(1, 16),
        fc2_w,
        fc2_b.reshape(1, 1),
    )
    return out


# speedup vs baseline: 1.9885x; 1.0198x over previous
import functools

import jax
import jax.numpy as jnp
from jax import lax
from jax.experimental import pallas as pl
from jax.experimental.pallas import tpu as pltpu
from jax.experimental.pallas import tpu_sc as plsc

N = 10000
E = 320000
NP = 10240
G = 64
NC = 2
NS = 16
NW = NC * NS
EPT = E // NW
CH = 128
NCHK = E // CH
CPW = NCHK // NW
CREM = NCHK - CPW * NW
RPT = NP // NS

_mesh = functools.partial(
    plsc.VectorSubcoreMesh, core_axis_name="c", subcore_axis_name="s"
)


def _deg_body(dst_hbm, out_hbm, idx_v, deg_v):
    c = lax.axis_index("c")
    s = lax.axis_index("s")
    wid = c * NS + s

    def zero(i, _):
        deg_v[pl.ds(i * 16, 16)] = jnp.zeros((16,), jnp.float32)
        return 0

    lax.fori_loop(0, NP // 16, zero, 0)

    pltpu.sync_copy(dst_hbm.at[pl.ds(wid * EPT, EPT)], idx_v)
    ones = jnp.ones((16,), jnp.float32)

    def body(j, _):
        idx = idx_v[pl.ds(j * 16, 16)]
        plsc.addupdate_scatter(deg_v, [idx], ones)
        return 0

    lax.fori_loop(0, EPT // 16, body, 0)
    pltpu.sync_copy(deg_v, out_hbm.at[wid])


def _deg_call(dst):
    return pl.kernel(
        _deg_body,
        out_type=jax.ShapeDtypeStruct((NW, NP), jnp.float32),
        mesh=_mesh(),
        scratch_types=[
            pltpu.VMEM((EPT,), jnp.int32),
            pltpu.VMEM((NP,), jnp.float32),
        ],
        compiler_params=pltpu.CompilerParams(needs_layout_passes=False),
    )(dst)


def _spmm_body(F, wide_scat, gs_hbm, src_hbm, dst1_hbm, out_hbm, sidx, didx, rows, acc, sem):
    c = lax.axis_index("c")
    s = lax.axis_index("s")
    wid = c * NS + s

    def zr(r, _):
        def zc(k, _):
            rows[r, pl.ds(k * 16, 16)] = jnp.zeros((16,), jnp.float32)
            return 0

        lax.fori_loop(0, F // 16, zc, 0)
        return 0

    lax.fori_loop(0, CH, zr, 0)
    for j in range(RPT // CH):
        pltpu.sync_copy(rows, acc.at[pl.ds(s * RPT + j * CH, CH)])
    plsc.subcore_barrier()

    base = wid * CPW + jnp.minimum(wid, CREM)
    n = jnp.where(wid < CREM, CPW + 1, CPW)
    pltpu.sync_copy(src_hbm.at[pl.ds(base, CPW + 1)], sidx)
    if not wide_scat:
        pltpu.sync_copy(dst1_hbm.at[pl.ds(base, CPW + 1)], didx)

    def edge(i, _):
        dma = pltpu.async_copy(gs_hbm.at[sidx.at[i]], rows, sem)
        if wide_scat:
            pltpu.sync_copy(dst1_hbm.at[pl.ds((base + i) * CH, CH)], didx)
            dma.wait()
            pltpu.sync_copy(rows, acc.at[didx], add=True)
        else:
            dma.wait()
            pltpu.sync_copy(rows, acc.at[didx.at[i]], add=True)
        return 0

    lax.fori_loop(0, n, edge, 0)
    plsc.subcore_barrier()
    pltpu.sync_copy(
        acc.at[pl.ds(s * RPT, RPT)], out_hbm.at[c, pl.ds(s * RPT, RPT)]
    )


def _spmm_call(F, wide_scat, tbl, src2d, dst):
    return pl.kernel(
        functools.partial(_spmm_body, F, wide_scat),
        out_type=jax.ShapeDtypeStruct((NC, NP, F), jnp.float32),
        mesh=_mesh(),
        scratch_types=[
            pltpu.VMEM((CPW + 1, CH), jnp.int32),
            pltpu.VMEM((CH,), jnp.int32)
            if wide_scat
            else pltpu.VMEM((CPW + 1, CH), jnp.int32),
            pltpu.VMEM((CH, F), jnp.float32),
            pltpu.VMEM_SHARED((NP, F), jnp.float32),
            pltpu.SemaphoreType.DMA,
        ],
        compiler_params=pltpu.CompilerParams(use_tc_tiling_on_sc=False),
    )(tbl, src2d, dst)


def _prep_body(degp_ref, x_ref, xs_ref, dc_ref):
    deg = 1.0 + jnp.sum(degp_ref[...], axis=0, keepdims=True)
    dinv = lax.rsqrt(jnp.maximum(deg, 1e-12))
    dc = jnp.reshape(dinv, (NP, 1))
    dc_ref[...] = dc
    xs_ref[...] = x_ref[...] * dc


def _mid_body(p0, p1, xs, d, w1, b1, w2, o):
    agg = d[...] * (p0[...] + p1[...] + xs[...])
    h1 = jnp.maximum(
        jnp.dot(agg, w1[...], preferred_element_type=jnp.float32) + b1[...], 0.0
    )
    g = jnp.dot(h1, w2[...], preferred_element_type=jnp.float32)
    o[...] = g * d[...]


def _head_body(q0, q1, gs, d, b2, bt, fc1w, fc1b, fc2w, fc2b, o):
    h2 = jnp.maximum(d[...] * (q0[...] + q1[...] + gs[...]) + b2[...], 0.0)
    gid = lax.broadcasted_iota(jnp.int32, (G, NP), 0)
    oh = (gid == bt[...]).astype(jnp.float32)
    psum = jnp.dot(oh, h2, preferred_element_type=jnp.float32)
    cnt = jnp.sum(oh, axis=1, keepdims=True)
    pooled = psum / jnp.maximum(cnt, 1.0)
    z = jnp.tanh(jnp.dot(pooled, fc1w[...], preferred_element_type=jnp.float32) + fc1b[...])
    zz = jnp.dot(z, fc2w[...], preferred_element_type=jnp.float32) + fc2b[...]
    o[...] = jax.nn.sigmoid(zz)


def kernel(x, edge_index, batch, W1, b1, W2, b2, fc1_w, fc1_b, fc2_w, fc2_b):
    f32 = jnp.float32
    src = edge_index[0].astype(jnp.int32)
    dst = edge_index[1].astype(jnp.int32)
    npad = (NCHK + CREM) * CH - E
    src2d = jnp.pad(src, (0, npad), constant_values=N).reshape(NCHK + CREM, CH)
    dst2d = jnp.pad(dst, (0, npad), constant_values=N).reshape(NCHK + CREM, CH)
    x_pad = jnp.pad(x.astype(f32), ((0, NP - N), (0, 0)))
    batch_pad = jnp.pad(
        batch.astype(jnp.int32), (0, NP - N), constant_values=2**20
    ).reshape(1, NP)

    degp = _deg_call(dst)

    RB = 1280
    xs, dinv_col = pl.pallas_call(
        _prep_body,
        out_shape=(
            jax.ShapeDtypeStruct((NP, 128), f32),
            jax.ShapeDtypeStruct((NP, 1), f32),
        ),
    )(degp, x_pad)

    P = _spmm_call(128, True, xs, src2d, dst)

    gs = pl.pallas_call(
        _mid_body,
        grid=(NP // RB,),
        in_specs=[
            pl.BlockSpec((RB, 128), lambda i: (i, 0)),
            pl.BlockSpec((RB, 128), lambda i: (i, 0)),
            pl.BlockSpec((RB, 128), lambda i: (i, 0)),
            pl.BlockSpec((RB, 1), lambda i: (i, 0)),
            pl.BlockSpec((128, 128), lambda i: (0, 0)),
            pl.BlockSpec((1, 128), lambda i: (0, 0)),
            pl.BlockSpec((128, 32), lambda i: (0, 0)),
        ],
        out_specs=pl.BlockSpec((RB, 32), lambda i: (i, 0)),
        out_shape=jax.ShapeDtypeStruct((NP, 32), f32),
    )(P[0], P[1], xs, dinv_col, W1, b1.reshape(1, 128), W2)

    Q = _spmm_call(32, False, gs, src2d, dst2d)

    out = pl.pallas_call(
        _head_body,
        out_shape=jax.ShapeDtypeStruct((G, 1), f32),
    )(
        Q[0],
        Q[1],
        gs,
        dinv_col,
        b2.reshape(1, 32),
        batch_pad,
        fc1_w,
        fc1_b.reshape(1, 16),
        fc2_w,
        fc2_b.reshape(1, 1),
    )
    return out


# speedup vs baseline: 2.7638x; 1.3898x over previous
import functools

import jax
import jax.numpy as jnp
from jax import lax
from jax.experimental import pallas as pl
from jax.experimental.pallas import tpu as pltpu
from jax.experimental.pallas import tpu_sc as plsc

N = 10000
E = 320000
NP = 10240
G = 64
NC = 2
NS = 16
NW = NC * NS
EPT = E // NW
CH = 128
NCHK = E // CH
CPW = NCHK // NW
CREM = NCHK - CPW * NW
RPT = NP // NS

_mesh = functools.partial(
    plsc.VectorSubcoreMesh, core_axis_name="c", subcore_axis_name="s"
)


def _deg_body(dst_hbm, out_hbm, idx_v, deg_v):
    c = lax.axis_index("c")
    s = lax.axis_index("s")
    wid = c * NS + s

    def zero(i, _):
        deg_v[pl.ds(i * 16, 16)] = jnp.zeros((16,), jnp.float32)
        return 0

    lax.fori_loop(0, NP // 16, zero, 0)

    pltpu.sync_copy(dst_hbm.at[pl.ds(wid * EPT, EPT)], idx_v)
    ones = jnp.ones((16,), jnp.float32)

    def body(j, _):
        idx = idx_v[pl.ds(j * 16, 16)]
        plsc.addupdate_scatter(deg_v, [idx], ones)
        return 0

    lax.fori_loop(0, EPT // 16, body, 0)
    pltpu.sync_copy(deg_v, out_hbm.at[wid])


def _deg_call(dst):
    return pl.kernel(
        _deg_body,
        out_type=jax.ShapeDtypeStruct((NW, NP), jnp.float32),
        mesh=_mesh(),
        scratch_types=[
            pltpu.VMEM((EPT,), jnp.int32),
            pltpu.VMEM((NP,), jnp.float32),
        ],
        compiler_params=pltpu.CompilerParams(needs_layout_passes=False),
    )(dst)


def _spmm_body(F, wide_scat, gs_hbm, src_hbm, dst1_hbm, out_hbm, sidx, didx, rows, acc, sem):
    c = lax.axis_index("c")
    s = lax.axis_index("s")
    wid = c * NS + s

    def zr(r, _):
        def zc(k, _):
            rows[0][r, pl.ds(k * 16, 16)] = jnp.zeros((16,), jnp.float32)
            return 0

        lax.fori_loop(0, F // 16, zc, 0)
        return 0

    lax.fori_loop(0, CH, zr, 0)
    for j in range(RPT // CH):
        pltpu.sync_copy(rows[0], acc.at[pl.ds(s * RPT + j * CH, CH)])
    plsc.subcore_barrier()

    base = wid * CPW + jnp.minimum(wid, CREM)
    n = jnp.where(wid < CREM, CPW + 1, CPW)
    pltpu.sync_copy(src_hbm.at[pl.ds(base, CPW + 1)], sidx)
    if not wide_scat:
        pltpu.sync_copy(dst1_hbm.at[pl.ds(base, CPW + 1)], didx)

    r0, r1 = rows

    def gstart(i, buf, k):
        pltpu.async_copy(gs_hbm.at[sidx.at[i]], buf, sem.at[k])

    def gwait(buf, k):
        pltpu.make_async_copy(gs_hbm.at[sidx.at[0]], buf, sem.at[k]).wait()

    def scat(i, buf, k):
        if wide_scat:
            pltpu.sync_copy(dst1_hbm.at[pl.ds((base + i) * CH, CH)], didx)
            gwait(buf, k)
            pltpu.sync_copy(buf, acc.at[didx], add=True)
        else:
            gwait(buf, k)
            pltpu.sync_copy(buf, acc.at[didx.at[i]], add=True)

    gstart(0, r0, 0)

    def edge(j, _):
        i0 = 2 * j
        gstart(i0 + 1, r1, 1)
        scat(i0, r0, 0)

        @pl.when(i0 + 2 < n)
        def _():
            gstart(i0 + 2, r0, 0)

        scat(i0 + 1, r1, 1)
        return 0

    lax.fori_loop(0, CPW // 2, edge, 0)

    @pl.when(CPW < n)
    def _():
        scat(CPW, r0, 0)

    plsc.subcore_barrier()
    pltpu.sync_copy(
        acc.at[pl.ds(s * RPT, RPT)], out_hbm.at[c, pl.ds(s * RPT, RPT)]
    )


def _spmm_call(F, wide_scat, tbl, src2d, dst):
    return pl.kernel(
        functools.partial(_spmm_body, F, wide_scat),
        out_type=jax.ShapeDtypeStruct((NC, NP, F), jnp.float32),
        mesh=_mesh(),
        scratch_types=[
            pltpu.VMEM((CPW + 1, CH), jnp.int32),
            pltpu.VMEM((CH,), jnp.int32)
            if wide_scat
            else pltpu.VMEM((CPW + 1, CH), jnp.int32),
            (
                pltpu.VMEM((CH, F), jnp.float32),
                pltpu.VMEM((CH, F), jnp.float32),
            ),
            pltpu.VMEM_SHARED((NP, F), jnp.float32),
            pltpu.SemaphoreType.DMA((2,)),
        ],
        compiler_params=pltpu.CompilerParams(use_tc_tiling_on_sc=False),
    )(tbl, src2d, dst)


def _prep_body(degp_ref, x_ref, xs_ref, dc_ref):
    deg = 1.0 + jnp.sum(degp_ref[...], axis=0, keepdims=True)
    dinv = lax.rsqrt(jnp.maximum(deg, 1e-12))
    dc = jnp.reshape(dinv, (NP, 1))
    dc_ref[...] = dc
    xs_ref[...] = x_ref[...] * dc


def _mid_body(p0, p1, xs, d, w1, b1, w2, o):
    agg = d[...] * (p0[...] + p1[...] + xs[...])
    h1 = jnp.maximum(
        jnp.dot(agg, w1[...], preferred_element_type=jnp.float32) + b1[...], 0.0
    )
    g = jnp.dot(h1, w2[...], preferred_element_type=jnp.float32)
    o[...] = g * d[...]


def _head_body(q0, q1, gs, d, b2, bt, fc1w, fc1b, fc2w, fc2b, o):
    h2 = jnp.maximum(d[...] * (q0[...] + q1[...] + gs[...]) + b2[...], 0.0)
    gid = lax.broadcasted_iota(jnp.int32, (G, NP), 0)
    oh = (gid == bt[...]).astype(jnp.float32)
    psum = jnp.dot(oh, h2, preferred_element_type=jnp.float32)
    cnt = jnp.sum(oh, axis=1, keepdims=True)
    pooled = psum / jnp.maximum(cnt, 1.0)
    z = jnp.tanh(jnp.dot(pooled, fc1w[...], preferred_element_type=jnp.float32) + fc1b[...])
    zz = jnp.dot(z, fc2w[...], preferred_element_type=jnp.float32) + fc2b[...]
    o[...] = jax.nn.sigmoid(zz)


def kernel(x, edge_index, batch, W1, b1, W2, b2, fc1_w, fc1_b, fc2_w, fc2_b):
    f32 = jnp.float32
    src = edge_index[0].astype(jnp.int32)
    dst = edge_index[1].astype(jnp.int32)
    npad = (NCHK + CREM) * CH - E
    src2d = jnp.pad(src, (0, npad), constant_values=N).reshape(NCHK + CREM, CH)
    dst2d = jnp.pad(dst, (0, npad), constant_values=N).reshape(NCHK + CREM, CH)
    x_pad = jnp.pad(x.astype(f32), ((0, NP - N), (0, 0)))
    batch_pad = jnp.pad(
        batch.astype(jnp.int32), (0, NP - N), constant_values=2**20
    ).reshape(1, NP)

    degp = _deg_call(dst)

    RB = 1280
    xs, dinv_col = pl.pallas_call(
        _prep_body,
        out_shape=(
            jax.ShapeDtypeStruct((NP, 128), f32),
            jax.ShapeDtypeStruct((NP, 1), f32),
        ),
    )(degp, x_pad)

    P = _spmm_call(128, True, xs, src2d, dst)

    gs = pl.pallas_call(
        _mid_body,
        grid=(NP // RB,),
        in_specs=[
            pl.BlockSpec((RB, 128), lambda i: (i, 0)),
            pl.BlockSpec((RB, 128), lambda i: (i, 0)),
            pl.BlockSpec((RB, 128), lambda i: (i, 0)),
            pl.BlockSpec((RB, 1), lambda i: (i, 0)),
            pl.BlockSpec((128, 128), lambda i: (0, 0)),
            pl.BlockSpec((1, 128), lambda i: (0, 0)),
            pl.BlockSpec((128, 32), lambda i: (0, 0)),
        ],
        out_specs=pl.BlockSpec((RB, 32), lambda i: (i, 0)),
        out_shape=jax.ShapeDtypeStruct((NP, 32), f32),
    )(P[0], P[1], xs, dinv_col, W1, b1.reshape(1, 128), W2)

    Q = _spmm_call(32, False, gs, src2d, dst2d)

    out = pl.pallas_call(
        _head_body,
        out_shape=jax.ShapeDtypeStruct((G, 1), f32),
    )(
        Q[0],
        Q[1],
        gs,
        dinv_col,
        b2.reshape(1, 32),
        batch_pad,
        fc1_w,
        fc1_b.reshape(1, 16),
        fc2_w,
        fc2_b.reshape(1, 1),
    )
    return out


# speedup vs baseline: 2.9467x; 1.0662x over previous
import functools

import jax
import jax.numpy as jnp
from jax import lax
from jax.experimental import pallas as pl
from jax.experimental.pallas import tpu as pltpu
from jax.experimental.pallas import tpu_sc as plsc

N = 10000
E = 320000
NP = 10240
G = 64
NC = 2
NS = 16
NW = NC * NS
EPT = E // NW
CH = 128
NCHK = E // CH
CPW = NCHK // NW
CREM = NCHK - CPW * NW
RPT = NP // NS

_mesh = functools.partial(
    plsc.VectorSubcoreMesh, core_axis_name="c", subcore_axis_name="s"
)


def _deg_body(dst_hbm, out_hbm, idx_v, deg_v):
    c = lax.axis_index("c")
    s = lax.axis_index("s")
    wid = c * NS + s

    def zero(i, _):
        deg_v[pl.ds(i * 16, 16)] = jnp.zeros((16,), jnp.float32)
        return 0

    lax.fori_loop(0, NP // 16, zero, 0)

    pltpu.sync_copy(dst_hbm.at[pl.ds(wid * EPT, EPT)], idx_v)
    ones = jnp.ones((16,), jnp.float32)

    def body(j, _):
        idx = idx_v[pl.ds(j * 16, 16)]
        plsc.addupdate_scatter(deg_v, [idx], ones)
        return 0

    lax.fori_loop(0, EPT // 16, body, 0)
    pltpu.sync_copy(deg_v, out_hbm.at[wid])


def _deg_call(dst):
    return pl.kernel(
        _deg_body,
        out_type=jax.ShapeDtypeStruct((NW, NP), jnp.float32),
        mesh=_mesh(),
        scratch_types=[
            pltpu.VMEM((EPT,), jnp.int32),
            pltpu.VMEM((NP,), jnp.float32),
        ],
        compiler_params=pltpu.CompilerParams(needs_layout_passes=False),
    )(dst)


def _spmm_body(F, wide_scat, gs_hbm, src_hbm, dst1_hbm, out_hbm, sidx, didx, rows, acc, sem):
    c = lax.axis_index("c")
    s = lax.axis_index("s")
    wid = c * NS + s

    def zr(r, _):
        def zc(k, _):
            rows[0][r, pl.ds(k * 16, 16)] = jnp.zeros((16,), jnp.float32)
            return 0

        lax.fori_loop(0, F // 16, zc, 0)
        return 0

    lax.fori_loop(0, CH, zr, 0)
    for j in range(RPT // CH):
        pltpu.sync_copy(rows[0], acc.at[pl.ds(s * RPT + j * CH, CH)])
    plsc.subcore_barrier()

    base = wid * CPW + jnp.minimum(wid, CREM)
    n = jnp.where(wid < CREM, CPW + 1, CPW)
    pltpu.sync_copy(src_hbm.at[pl.ds(base, CPW + 1)], sidx)
    if not wide_scat:
        pltpu.sync_copy(dst1_hbm.at[pl.ds(base, CPW + 1)], didx)

    NB = len(rows)

    def gstart(i, k):
        pltpu.async_copy(gs_hbm.at[sidx.at[i]], rows[k], sem.at[k])

    def gwait(k):
        pltpu.make_async_copy(gs_hbm.at[sidx.at[0]], rows[k], sem.at[k]).wait()

    def scat(i, k):
        if wide_scat:
            pltpu.sync_copy(dst1_hbm.at[pl.ds((base + i) * CH, CH)], didx)
            gwait(k)
            pltpu.sync_copy(rows[k], acc.at[didx], add=True)
        else:
            gwait(k)
            pltpu.sync_copy(rows[k], acc.at[didx.at[i]], add=True)

    for b in range(NB - 1):
        gstart(b, b)

    def edge(j, _):
        i0 = NB * j

        for b in range(NB):
            i = i0 + b
            gstart(i + NB - 1, (b + NB - 1) % NB)
            scat(i, b)
        return 0

    NGRP = (CPW - NB + 1) // NB
    lax.fori_loop(0, NGRP, edge, 0)
    for t in range(NGRP * NB, CPW):
        b = t % NB

        @pl.when(t + NB - 1 < n)
        def _(t=t, b=(t + NB - 1) % NB):
            gstart(t + NB - 1, b)

        scat(t, b)

    @pl.when(CPW < n)
    def _():
        scat(CPW, CPW % NB)

    plsc.subcore_barrier()
    pltpu.sync_copy(
        acc.at[pl.ds(s * RPT, RPT)], out_hbm.at[c, pl.ds(s * RPT, RPT)]
    )


def _spmm_call(F, NB, wide_scat, tbl, src2d, dst):
    return pl.kernel(
        functools.partial(_spmm_body, F, wide_scat),
        out_type=jax.ShapeDtypeStruct((NC, NP, F), jnp.float32),
        mesh=_mesh(),
        scratch_types=[
            pltpu.VMEM((CPW + 1, CH), jnp.int32),
            pltpu.VMEM((CH,), jnp.int32)
            if wide_scat
            else pltpu.VMEM((CPW + 1, CH), jnp.int32),
            tuple(pltpu.VMEM((CH, F), jnp.float32) for _ in range(NB)),
            pltpu.VMEM_SHARED((NP, F), jnp.float32),
            pltpu.SemaphoreType.DMA((NB,)),
        ],
        compiler_params=pltpu.CompilerParams(use_tc_tiling_on_sc=False),
    )(tbl, src2d, dst)


def _prep_body(degp_ref, x_ref, xs_ref, dc_ref):
    deg = 1.0 + jnp.sum(degp_ref[...], axis=0, keepdims=True)
    dinv = lax.rsqrt(jnp.maximum(deg, 1e-12))
    dc = jnp.reshape(dinv, (NP, 1))
    dc_ref[...] = dc
    xs_ref[...] = x_ref[...] * dc


def _mid_body(p0, p1, xs, d, w1, b1, w2, o):
    agg = d[...] * (p0[...] + p1[...] + xs[...])
    h1 = jnp.maximum(
        jnp.dot(agg, w1[...], preferred_element_type=jnp.float32) + b1[...], 0.0
    )
    g = jnp.dot(h1, w2[...], preferred_element_type=jnp.float32)
    o[...] = g * d[...]


def _head_body(q0, q1, gs, d, b2, bt, fc1w, fc1b, fc2w, fc2b, o):
    h2 = jnp.maximum(d[...] * (q0[...] + q1[...] + gs[...]) + b2[...], 0.0)
    gid = lax.broadcasted_iota(jnp.int32, (G, NP), 0)
    oh = (gid == bt[...]).astype(jnp.float32)
    psum = jnp.dot(oh, h2, preferred_element_type=jnp.float32)
    cnt = jnp.sum(oh, axis=1, keepdims=True)
    pooled = psum / jnp.maximum(cnt, 1.0)
    z = jnp.tanh(jnp.dot(pooled, fc1w[...], preferred_element_type=jnp.float32) + fc1b[...])
    zz = jnp.dot(z, fc2w[...], preferred_element_type=jnp.float32) + fc2b[...]
    o[...] = jax.nn.sigmoid(zz)


def kernel(x, edge_index, batch, W1, b1, W2, b2, fc1_w, fc1_b, fc2_w, fc2_b):
    f32 = jnp.float32
    src = edge_index[0].astype(jnp.int32)
    dst = edge_index[1].astype(jnp.int32)
    npad = (NCHK + CREM) * CH - E
    src2d = jnp.pad(src, (0, npad), constant_values=N).reshape(NCHK + CREM, CH)
    dst2d = jnp.pad(dst, (0, npad), constant_values=N).reshape(NCHK + CREM, CH)
    x_pad = jnp.pad(x.astype(f32), ((0, NP - N), (0, 0)))
    batch_pad = jnp.pad(
        batch.astype(jnp.int32), (0, NP - N), constant_values=2**20
    ).reshape(1, NP)

    degp = _deg_call(dst)

    RB = 1280
    xs, dinv_col = pl.pallas_call(
        _prep_body,
        out_shape=(
            jax.ShapeDtypeStruct((NP, 128), f32),
            jax.ShapeDtypeStruct((NP, 1), f32),
        ),
    )(degp, x_pad)

    P = _spmm_call(128, 2, True, xs, src2d, dst)

    gs = pl.pallas_call(
        _mid_body,
        grid=(NP // RB,),
        in_specs=[
            pl.BlockSpec((RB, 128), lambda i: (i, 0)),
            pl.BlockSpec((RB, 128), lambda i: (i, 0)),
            pl.BlockSpec((RB, 128), lambda i: (i, 0)),
            pl.BlockSpec((RB, 1), lambda i: (i, 0)),
            pl.BlockSpec((128, 128), lambda i: (0, 0)),
            pl.BlockSpec((1, 128), lambda i: (0, 0)),
            pl.BlockSpec((128, 32), lambda i: (0, 0)),
        ],
        out_specs=pl.BlockSpec((RB, 32), lambda i: (i, 0)),
        out_shape=jax.ShapeDtypeStruct((NP, 32), f32),
    )(P[0], P[1], xs, dinv_col, W1, b1.reshape(1, 128), W2)

    Q = _spmm_call(32, 4, False, gs, src2d, dst2d)

    out = pl.pallas_call(
        _head_body,
        out_shape=jax.ShapeDtypeStruct((G, 1), f32),
    )(
        Q[0],
        Q[1],
        gs,
        dinv_col,
        b2.reshape(1, 32),
        batch_pad,
        fc1_w,
        fc1_b.reshape(1, 16),
        fc2_w,
        fc2_b.reshape(1, 1),
    )
    return out


# speedup vs baseline: 3.0010x; 1.0184x over previous
import functools

import jax
import jax.numpy as jnp
from jax import lax
from jax.experimental import pallas as pl
from jax.experimental.pallas import tpu as pltpu
from jax.experimental.pallas import tpu_sc as plsc

N = 10000
E = 320000
NP = 10240
G = 64
NC = 2
NS = 16
NW = NC * NS
EPT = E // NW
CH = 128
NCHK = E // CH
CPW = NCHK // NW
CREM = NCHK - CPW * NW
RPT = NP // NS

_mesh = functools.partial(
    plsc.VectorSubcoreMesh, core_axis_name="c", subcore_axis_name="s"
)


def _deg_body(dst_hbm, out_hbm, idx_v, deg_v):
    c = lax.axis_index("c")
    s = lax.axis_index("s")
    wid = c * NS + s

    def zero(i, _):
        deg_v[pl.ds(i * 16, 16)] = jnp.zeros((16,), jnp.float32)
        return 0

    lax.fori_loop(0, NP // 16, zero, 0)

    pltpu.sync_copy(dst_hbm.at[pl.ds(wid * EPT, EPT)], idx_v)
    ones = jnp.ones((16,), jnp.float32)

    def body(j, _):
        idx = idx_v[pl.ds(j * 16, 16)]
        plsc.addupdate_scatter(deg_v, [idx], ones)
        return 0

    lax.fori_loop(0, EPT // 16, body, 0)
    pltpu.sync_copy(deg_v, out_hbm.at[wid])


def _deg_call(dst):
    return pl.kernel(
        _deg_body,
        out_type=jax.ShapeDtypeStruct((NW, NP), jnp.float32),
        mesh=_mesh(),
        scratch_types=[
            pltpu.VMEM((EPT,), jnp.int32),
            pltpu.VMEM((NP,), jnp.float32),
        ],
        compiler_params=pltpu.CompilerParams(needs_layout_passes=False),
    )(dst)


def _spmm_body(F, wide_scat, gs_hbm, src_hbm, dst1_hbm, out_hbm, sidx, didx, rows, acc, sem):
    c = lax.axis_index("c")
    s = lax.axis_index("s")
    wid = c * NS + s

    def zr(r, _):
        def zc(k, _):
            rows[0][r, pl.ds(k * 16, 16)] = jnp.zeros((16,), jnp.float32)
            return 0

        lax.fori_loop(0, F // 16, zc, 0)
        return 0

    lax.fori_loop(0, CH, zr, 0)
    for j in range(RPT // CH):
        pltpu.sync_copy(rows[0], acc.at[pl.ds(s * RPT + j * CH, CH)])
    plsc.subcore_barrier()

    base = wid * CPW + jnp.minimum(wid, CREM)
    n = jnp.where(wid < CREM, CPW + 1, CPW)
    pltpu.sync_copy(src_hbm.at[pl.ds(base, CPW + 1)], sidx)
    if not wide_scat:
        pltpu.sync_copy(dst1_hbm.at[pl.ds(base, CPW + 1)], didx)

    NB = len(rows)

    def gstart(i, k):
        pltpu.async_copy(gs_hbm.at[sidx.at[i]], rows[k], sem.at[k])

    def gwait(k):
        pltpu.make_async_copy(gs_hbm.at[sidx.at[0]], rows[k], sem.at[k]).wait()

    def scat(i, k):
        if wide_scat:
            pltpu.sync_copy(dst1_hbm.at[pl.ds((base + i) * CH, CH)], didx)
            gwait(k)
            pltpu.sync_copy(rows[k], acc.at[didx], add=True)
        else:
            gwait(k)
            pltpu.sync_copy(rows[k], acc.at[didx.at[i]], add=True)

    for b in range(NB - 1):
        gstart(b, b)

    def edge(j, _):
        i0 = NB * j

        for b in range(NB):
            i = i0 + b
            gstart(i + NB - 1, (b + NB - 1) % NB)
            scat(i, b)
        return 0

    NGRP = (CPW - NB + 1) // NB
    lax.fori_loop(0, NGRP, edge, 0)
    for t in range(NGRP * NB, CPW):
        b = t % NB

        @pl.when(t + NB - 1 < n)
        def _(t=t, b=(t + NB - 1) % NB):
            gstart(t + NB - 1, b)

        scat(t, b)

    @pl.when(CPW < n)
    def _():
        scat(CPW, CPW % NB)

    plsc.subcore_barrier()
    pltpu.sync_copy(
        acc.at[pl.ds(s * RPT, RPT)], out_hbm.at[c, pl.ds(s * RPT, RPT)]
    )


def _spmm_call(F, NB, wide_scat, tbl, src2d, dst):
    return pl.kernel(
        functools.partial(_spmm_body, F, wide_scat),
        out_type=jax.ShapeDtypeStruct((NC, NP, F), jnp.float32),
        mesh=_mesh(),
        scratch_types=[
            pltpu.VMEM((CPW + 1, CH), jnp.int32),
            pltpu.VMEM((CH,), jnp.int32)
            if wide_scat
            else pltpu.VMEM((CPW + 1, CH), jnp.int32),
            tuple(pltpu.VMEM((CH, F), jnp.float32) for _ in range(NB)),
            pltpu.VMEM_SHARED((NP, F), jnp.float32),
            pltpu.SemaphoreType.DMA((NB,)),
        ],
        compiler_params=pltpu.CompilerParams(use_tc_tiling_on_sc=False),
    )(tbl, src2d, dst)


def _prep_body(degp_ref, x_ref, xs_ref, dc_ref):
    deg = 1.0 + jnp.sum(degp_ref[...], axis=0, keepdims=True)
    dinv = lax.rsqrt(jnp.maximum(deg, 1e-12))
    dc = jnp.reshape(dinv, (NP, 1))
    dc_ref[...] = dc
    xs_ref[pl.ds(0, N), :] = x_ref[...] * dc[:N, :]
    xs_ref[pl.ds(N, NP - N), :] = jnp.zeros((NP - N, 128), jnp.float32)


def _mid_body(p0, p1, xs, d, w1, b1, w2, o):
    agg = d[...] * (p0[...] + p1[...] + xs[...])
    h1 = jnp.maximum(
        jnp.dot(agg, w1[...], preferred_element_type=jnp.float32) + b1[...], 0.0
    )
    g = jnp.dot(h1, w2[...], preferred_element_type=jnp.float32)
    o[...] = g * d[...]


def _head_body(q0, q1, gs, d, b2, bt, fc1w, fc1b, fc2w, fc2b, o):
    h2 = jnp.maximum(d[...] * (q0[...] + q1[...] + gs[...]) + b2[...], 0.0)
    gid = lax.broadcasted_iota(jnp.int32, (G, NP), 0)
    oh = (gid == bt[...]).astype(jnp.float32)
    psum = jnp.dot(oh, h2, preferred_element_type=jnp.float32)
    cnt = jnp.sum(oh, axis=1, keepdims=True)
    pooled = psum / jnp.maximum(cnt, 1.0)
    z = jnp.tanh(jnp.dot(pooled, fc1w[...], preferred_element_type=jnp.float32) + fc1b[...])
    zz = jnp.dot(z, fc2w[...], preferred_element_type=jnp.float32) + fc2b[...]
    o[...] = jax.nn.sigmoid(zz)


def kernel(x, edge_index, batch, W1, b1, W2, b2, fc1_w, fc1_b, fc2_w, fc2_b):
    f32 = jnp.float32
    src = edge_index[0].astype(jnp.int32)
    dst = edge_index[1].astype(jnp.int32)
    npad = (NCHK + CREM) * CH - E
    src2d = jnp.pad(src, (0, npad), constant_values=N).reshape(NCHK + CREM, CH)
    dst2d = jnp.pad(dst, (0, npad), constant_values=N).reshape(NCHK + CREM, CH)
    batch_pad = jnp.pad(
        batch.astype(jnp.int32), (0, NP - N), constant_values=2**20
    ).reshape(1, NP)

    degp = _deg_call(dst)

    RB = 1280
    xs, dinv_col = pl.pallas_call(
        _prep_body,
        out_shape=(
            jax.ShapeDtypeStruct((NP, 128), f32),
            jax.ShapeDtypeStruct((NP, 1), f32),
        ),
    )(degp, x.astype(f32))

    P = _spmm_call(128, 2, True, xs, src2d, dst)

    gs = pl.pallas_call(
        _mid_body,
        grid=(NP // RB,),
        in_specs=[
            pl.BlockSpec((RB, 128), lambda i: (i, 0)),
            pl.BlockSpec((RB, 128), lambda i: (i, 0)),
            pl.BlockSpec((RB, 128), lambda i: (i, 0)),
            pl.BlockSpec((RB, 1), lambda i: (i, 0)),
            pl.BlockSpec((128, 128), lambda i: (0, 0)),
            pl.BlockSpec((1, 128), lambda i: (0, 0)),
            pl.BlockSpec((128, 32), lambda i: (0, 0)),
        ],
        out_specs=pl.BlockSpec((RB, 32), lambda i: (i, 0)),
        out_shape=jax.ShapeDtypeStruct((NP, 32), f32),
    )(P[0], P[1], xs, dinv_col, W1, b1.reshape(1, 128), W2)

    Q = _spmm_call(32, 8, False, gs, src2d, dst2d)

    out = pl.pallas_call(
        _head_body,
        out_shape=jax.ShapeDtypeStruct((G, 1), f32),
    )(
        Q[0],
        Q[1],
        gs,
        dinv_col,
        b2.reshape(1, 32),
        batch_pad,
        fc1_w,
        fc1_b.reshape(1, 16),
        fc2_w,
        fc2_b.reshape(1, 1),
    )
    return out
